# baseline (speedup 1.0000x reference)
import functools

import jax
import jax.numpy as jnp
from jax import lax
from jax.experimental import pallas as pl
from jax.experimental.pallas import tpu as pltpu
from jax.experimental.pallas import tpu_sc as plsc

D_MODEL = 2048
N_ROWS = 4 * 4096
TC_ROWS = 8192
SC_ROWS = N_ROWS - TC_ROWS
NUM_WORKERS = 32
ROWS_PER_WORKER = SC_ROWS // NUM_WORKERS
TC_BLOCK = 512

_mesh = plsc.VectorSubcoreMesh(core_axis_name="c", subcore_axis_name="s")


@functools.partial(
    pl.kernel,
    mesh=_mesh,
    out_type=jax.ShapeDtypeStruct((SC_ROWS, D_MODEL), jnp.float32),
    scratch_types=[
        pltpu.VMEM((2, D_MODEL), jnp.float32),
        pltpu.VMEM((ROWS_PER_WORKER,), jnp.int32),
        pltpu.SemaphoreType.DMA,
    ],
)
def _embed_sc(idx_hbm, table_hbm, out_hbm, table_v, idx_v, sem):
    wid = lax.axis_index("s") * 2 + lax.axis_index("c")
    base = wid * ROWS_PER_WORKER
    pltpu.sync_copy(table_hbm, table_v)
    pltpu.sync_copy(idx_hbm.at[pl.ds(base, ROWS_PER_WORKER)], idx_v)

    @pl.loop(0, ROWS_PER_WORKER // 16)
    def _groups(g):
        off = g * 16
        xv = idx_v[pl.ds(off, 16)]
        for l in range(16):
            pltpu.async_copy(
                table_v.at[xv[l]], out_hbm.at[base + off + l], sem
            )

    slab = out_hbm.at[pl.ds(base, ROWS_PER_WORKER)]
    pltpu.make_async_copy(slab, slab, sem).wait()


def _tc_body(x_ref, w_ref, o_ref):
    xf = x_ref[0, 0].astype(jnp.float32).reshape(TC_BLOCK, 1)
    w0 = w_ref[0].reshape(1, D_MODEL)
    w1 = w_ref[1].reshape(1, D_MODEL)
    o_ref[...] = w0 + xf * (w1 - w0)


_embed_tc = pl.pallas_call(
    _tc_body,
    grid=(TC_ROWS // TC_BLOCK,),
    in_specs=[
        pl.BlockSpec((1, 1, TC_BLOCK), lambda i: (i, 0, 0)),
        pl.BlockSpec((2, D_MODEL), lambda i: (0, 0)),
    ],
    out_specs=pl.BlockSpec((TC_BLOCK, D_MODEL), lambda i: (i, 0)),
    out_shape=jax.ShapeDtypeStruct((TC_ROWS, D_MODEL), jnp.float32),
)


def kernel(x, emb_weight):
    idx = x.reshape(-1).astype(jnp.int32)
    out_tc = _embed_tc(
        idx[:TC_ROWS].reshape(TC_ROWS // TC_BLOCK, 1, TC_BLOCK), emb_weight)
    out_sc = _embed_sc(idx[TC_ROWS:], emb_weight)
    out = jnp.concatenate([out_tc, out_sc], axis=0)
    return out.reshape(x.shape + (D_MODEL,))


# speedup vs baseline: 2.2961x; 2.2961x over previous
import functools

import jax
import jax.numpy as jnp
from jax import lax
from jax.experimental import pallas as pl
from jax.experimental.pallas import tpu as pltpu
from jax.experimental.pallas import tpu_sc as plsc

BATCH = 4
SEQ = 4096
D_MODEL = 2048
NUM_WORKERS = 32
ROWS_PER_WORKER = BATCH * SEQ // NUM_WORKERS
SLABS_PER_BATCH = SEQ // ROWS_PER_WORKER

_mesh = plsc.VectorSubcoreMesh(core_axis_name="c", subcore_axis_name="s")


@functools.partial(
    pl.kernel,
    mesh=_mesh,
    out_type=jax.ShapeDtypeStruct((BATCH, SEQ, D_MODEL), jnp.float32),
    scratch_types=[
        pltpu.VMEM((2, D_MODEL), jnp.float32),
        pltpu.VMEM((ROWS_PER_WORKER,), jnp.int32),
        pltpu.SemaphoreType.DMA,
    ],
)
def _embed_sc(x_hbm, table_hbm, out_hbm, table_v, idx_v, sem):
    wid = lax.axis_index("s") * 2 + lax.axis_index("c")
    b = wid // SLABS_PER_BATCH
    s0 = (wid % SLABS_PER_BATCH) * ROWS_PER_WORKER
    pltpu.sync_copy(table_hbm, table_v)
    pltpu.sync_copy(x_hbm.at[b, pl.ds(s0, ROWS_PER_WORKER)], idx_v)

    @pl.loop(0, ROWS_PER_WORKER // 16)
    def _groups(g):
        off = g * 16
        xv = idx_v[pl.ds(off, 16)]
        for l in range(16):
            pltpu.async_copy(
                table_v.at[xv[l]], out_hbm.at[b, s0 + off + l], sem
            )

    slab = out_hbm.at[b, pl.ds(s0, ROWS_PER_WORKER)]
    pltpu.make_async_copy(slab, slab, sem).wait()


def kernel(x, emb_weight):
    return _embed_sc(x.astype(jnp.int32), emb_weight)
